# baseline (speedup 1.0000x reference)
import functools

import jax
import jax.numpy as jnp
from jax import lax
from jax.experimental import pallas as pl
from jax.experimental.pallas import tpu as pltpu
from jax.experimental.pallas import tpu_sc as plsc

N = 50000
D = 128
E = 500000
OUT = 64

NC = 2
NS = 16
CW = 64
NCP = D // CW

RPT = 3136
N_PAD = RPT * NS
EPT = 32768
EP = EPT * NS
S = 4096
NCHUNK = EPT // S
SUB = 128
NSUB = S // SUB
NBUF = 4


def _sc_segment_sums(xcat, src_all, dst_all, w_all):
    mesh = plsc.VectorSubcoreMesh(
        core_axis_name="c", subcore_axis_name="s", num_cores=NC, num_subcores=NS
    )

    @functools.partial(
        pl.kernel,
        out_type=jax.ShapeDtypeStruct((4 * NCP * N_PAD, CW), jnp.bfloat16),
        mesh=mesh,
        scratch_types=[
            pltpu.VMEM((S,), jnp.int32),
            pltpu.VMEM((NSUB, SUB), jnp.int32),
            pltpu.VMEM((S,), jnp.float32),
            [pltpu.VMEM((SUB, CW), jnp.bfloat16)] * NBUF,
            pltpu.VMEM((56, CW), jnp.bfloat16),
            pltpu.VMEM_SHARED((N_PAD, CW), jnp.bfloat16),
            [pltpu.SemaphoreType.DMA] * NBUF,
            [pltpu.SemaphoreType.DMA] * NBUF,
        ],
        compiler_params=pltpu.CompilerParams(use_tc_tiling_on_sc=False,
                                            needs_layout_passes=False),
    )
    def k(xcat_hbm, src_hbm, dst_hbm, w_hbm, out_hbm,
          src_v, dst_v, w_v, rows, zbuf_v, acc_sh, gsems, ssems):
        core = lax.axis_index("c")
        sub = lax.axis_index("s")

        zv = jnp.zeros((32,), jnp.bfloat16)
        def zb(i, _):
            zbuf_v[i, pl.ds(0, 32)] = zv
            zbuf_v[i, pl.ds(32, 32)] = zv
            return 0
        lax.fori_loop(0, 56, zb, 0)

        def fire_gather(s, b):
            pltpu.async_copy(
                xcat_hbm.at[src_v.at[pl.ds(pl.multiple_of(s * SUB, SUB), SUB)]],
                rows[b], gsems[b])

        def wait_gather(s, b):
            pltpu.make_async_copy(
                xcat_hbm.at[src_v.at[pl.ds(pl.multiple_of(s * SUB, SUB), SUB)]],
                rows[b], gsems[b]).wait()

        def fire_scatter(s, b):
            pltpu.async_copy(rows[b], acc_sh.at[dst_v.at[s]], ssems[b],
                             add=True)

        def wait_scatter(s, b):
            pltpu.make_async_copy(rows[b], acc_sh.at[dst_v.at[s]],
                                  ssems[b]).wait()

        def scale(b, s):
            rv = rows[b]
            def body(gi, _):
                w0 = pl.multiple_of(s * SUB + gi * 16, 16)
                wv = w_v[pl.ds(w0, 16)]
                for t in range(16):
                    w = wv[t]
                    r = gi * 16 + t
                    for half in (0, 32):
                        v = rv[r, pl.ds(half, 32)]
                        va, vb = plsc.unpack(
                            v, format=plsc.PackFormat.INTERLEAVED)
                        rv[r, pl.ds(half, 32)] = plsc.pack(
                            va * w, vb * w,
                            format=plsc.PackFormat.INTERLEAVED)
                return 0
            lax.fori_loop(0, SUB // 16, body, 0)

        def one_pass(p, _):
            et = core * 2 + p // NCP
            c = p % NCP
            def zstripe(z, _):
                pltpu.sync_copy(
                    zbuf_v, acc_sh.at[pl.ds(sub * RPT + z * 56, 56)])
                return 0
            lax.fori_loop(0, RPT // 56, zstripe, 0)
            plsc.subcore_barrier()

            def chunk(ch, _):
                f0 = et * EP + sub * EPT + ch * S
                pltpu.sync_copy(src_hbm.at[pl.ds(f0, S)], src_v)
                pltpu.sync_copy(w_hbm.at[pl.ds(f0, S)], w_v)
                drow = pl.multiple_of(f0 // SUB, 8)
                pltpu.sync_copy(dst_hbm.at[pl.ds(drow, NSUB)], dst_v)
                boff = core * (N * NCP) + c
                def adj(t, _):
                    v = src_v[pl.ds(t * 16, 16)]
                    src_v[pl.ds(t * 16, 16)] = v * NCP + boff
                    return 0
                lax.fori_loop(0, S // 16, adj, 0)

                for b in range(NBUF):
                    fire_gather(b, b)

                def step(g, _):
                    for b in range(NBUF):
                        s = g * NBUF + b
                        wait_gather(s, b)
                        scale(b, s)
                        fire_scatter(s, b)
                        wait_scatter(s, b)
                        fire_gather(s + NBUF, b)
                    return 0
                lax.fori_loop(0, NSUB // NBUF - 1, step, 0)

                for b in range(NBUF):
                    s = NSUB - NBUF + b
                    wait_gather(s, b)
                    scale(b, s)
                    pltpu.sync_copy(rows[b], acc_sh.at[dst_v.at[s]], add=True)
                return 0

            lax.fori_loop(0, NCHUNK, chunk, 0)
            plsc.subcore_barrier()
            obase = (et * NCP + c) * N_PAD + sub * RPT
            pltpu.sync_copy(acc_sh.at[pl.ds(sub * RPT, RPT)],
                            out_hbm.at[pl.ds(obase, RPT)])
            return 0

        lax.fori_loop(0, 2 * NCP, one_pass, 0)

    return k(xcat, src_all, dst_all, w_all)


def _dense_stage(aggs, x, wa, wb, wroot, bvec, wout, bout, et_a, et_b):
    BN = 1024
    grid = (N_PAD // BN,)

    def body(aggA, aggB, x_r, wa_r, wb_r, wr_r, b_r, wo_r, bo_r, o_r):
        h = jnp.dot(x_r[...], wr_r[...], preferred_element_type=jnp.float32)
        h = h + b_r[...]
        for c in range(NCP):
            h = h + jnp.dot(aggA[0, c], wa_r[c],
                            preferred_element_type=jnp.float32)
            h = h + jnp.dot(aggB[0, c], wb_r[c],
                            preferred_element_type=jnp.float32)
        h = jnp.maximum(h, 0.0)
        o = jnp.dot(h, wo_r[...], preferred_element_type=jnp.float32)
        o_r[...] = jnp.maximum(o + bo_r[...], 0.0)

    f = pl.pallas_call(
        body,
        grid=grid,
        in_specs=[
            pl.BlockSpec((1, NCP, BN, CW), lambda i, _e=et_a: (_e, 0, i, 0)),
            pl.BlockSpec((1, NCP, BN, CW), lambda i, _e=et_b: (_e, 0, i, 0)),
            pl.BlockSpec((BN, D), lambda i: (i, 0)),
            pl.BlockSpec((NCP, CW, D), lambda i: (0, 0, 0)),
            pl.BlockSpec((NCP, CW, D), lambda i: (0, 0, 0)),
            pl.BlockSpec((D, D), lambda i: (0, 0)),
            pl.BlockSpec((1, D), lambda i: (0, 0)),
            pl.BlockSpec((D, OUT), lambda i: (0, 0)),
            pl.BlockSpec((1, OUT), lambda i: (0, 0)),
        ],
        out_specs=pl.BlockSpec((BN, OUT), lambda i: (i, 0)),
        out_shape=jax.ShapeDtypeStruct((N, OUT), jnp.float32),
    )
    return f(aggs, aggs, x, wa, wb, wroot, bvec, wout, bout)


def kernel(x_general, x_rainfall, ei_gen_to_gen, ei_gen_to_rain,
           ei_rain_to_gen, ei_rain_to_rain, ew_gen_to_gen, ew_gen_to_rain,
           ew_rain_to_gen, ew_rain_to_rain, params):
    f32 = jnp.float32
    bf16 = jnp.bfloat16

    xcat = jnp.concatenate([x_general, x_rainfall], axis=0).astype(
        bf16).reshape(NC * NCP * N, CW)

    pad = EP - E
    srcs, dsts, ws = [], [], []
    for ei, ew in ((ei_gen_to_gen, ew_gen_to_gen),
                   (ei_gen_to_rain, ew_gen_to_rain),
                   (ei_rain_to_gen, ew_rain_to_gen),
                   (ei_rain_to_rain, ew_rain_to_rain)):
        srcs.append(jnp.concatenate([ei[0], jnp.zeros((pad,), jnp.int32)]))
        dsts.append(jnp.concatenate([ei[1], jnp.full((pad,), N, jnp.int32)]))
        ws.append(jnp.concatenate([ew, jnp.zeros((pad,), f32)]))
    src_all = jnp.concatenate(srcs)
    dst_all = jnp.concatenate(dsts).reshape(4 * EP // SUB, SUB)
    w_all = jnp.concatenate(ws)

    aggs = _sc_segment_sums(xcat, src_all, dst_all, w_all)
    aggs = aggs.reshape(4, NCP, N_PAD, CW)

    lp = params['layers'][-1]

    def rel_w(p):
        return p['W_rel'].T.reshape(NCP, CW, D).astype(bf16)

    outs = []
    for et_a, et_b, pa, pb, x, lin in (
            (0, 2, lp['gg'], lp['rg'], x_general, params['lin_general']),
            (1, 3, lp['gr'], lp['rr'], x_rainfall, params['lin_rainfall'])):
        wroot = (pa['W_root'] + pb['W_root']).T
        bvec = (pa['b_rel'] + pb['b_rel']).reshape(1, D)
        outs.append(_dense_stage(aggs, x, rel_w(pa), rel_w(pb), wroot, bvec,
                                 lin['W'].T, lin['b'].reshape(1, OUT),
                                 et_a, et_b))

    return (outs[0], outs[1])


# speedup vs baseline: 1.0110x; 1.0110x over previous
import functools

import jax
import jax.numpy as jnp
from jax import lax
from jax.experimental import pallas as pl
from jax.experimental.pallas import tpu as pltpu
from jax.experimental.pallas import tpu_sc as plsc

N = 50000
D = 128
E = 500000
OUT = 64

NC = 2
NS = 16
CW = 64
NCP = D // CW

RPT = 3136
N_PAD = RPT * NS
EPT = 32768
EP = EPT * NS
S = 4096
NCHUNK = EPT // S
SUB = 128
NSUB = S // SUB
NBUF = 4


def _sc_segment_sums(xcat, src_all, dst_all, w_all, zrows):
    mesh = plsc.VectorSubcoreMesh(
        core_axis_name="c", subcore_axis_name="s", num_cores=NC, num_subcores=NS
    )

    @functools.partial(
        pl.kernel,
        out_type=jax.ShapeDtypeStruct((4 * NCP * N_PAD, CW), jnp.bfloat16),
        mesh=mesh,
        scratch_types=[
            pltpu.VMEM((S,), jnp.int32),
            pltpu.VMEM((NSUB, SUB), jnp.int32),
            pltpu.VMEM((S,), jnp.float32),
            [pltpu.VMEM((SUB, CW), jnp.bfloat16)] * NBUF,
            pltpu.VMEM_SHARED((N_PAD, CW), jnp.bfloat16),
            [pltpu.SemaphoreType.DMA] * NBUF,
            [pltpu.SemaphoreType.DMA] * NBUF,
        ],
        compiler_params=pltpu.CompilerParams(use_tc_tiling_on_sc=False,
                                            needs_layout_passes=False),
    )
    def k(xcat_hbm, src_hbm, dst_hbm, w_hbm, z_hbm, out_hbm,
          src_v, dst_v, w_v, rows, acc_sh, gsems, ssems):
        core = lax.axis_index("c")
        sub = lax.axis_index("s")

        def fire_gather(s, b):
            pltpu.async_copy(
                xcat_hbm.at[src_v.at[pl.ds(pl.multiple_of(s * SUB, SUB), SUB)]],
                rows[b], gsems[b])

        def wait_gather(s, b):
            pltpu.make_async_copy(
                xcat_hbm.at[src_v.at[pl.ds(pl.multiple_of(s * SUB, SUB), SUB)]],
                rows[b], gsems[b]).wait()

        def fire_scatter(s, b):
            pltpu.async_copy(rows[b], acc_sh.at[dst_v.at[s]], ssems[b],
                             add=True)

        def wait_scatter(s, b):
            pltpu.make_async_copy(rows[b], acc_sh.at[dst_v.at[s]],
                                  ssems[b]).wait()

        def scale(b, s):
            rv = rows[b]
            def body(gi, _):
                w0 = pl.multiple_of(s * SUB + gi * 16, 16)
                wv = w_v[pl.ds(w0, 16)]
                for t in range(16):
                    w = wv[t]
                    r = gi * 16 + t
                    for half in (0, 32):
                        v = rv[r, pl.ds(half, 32)]
                        va, vb = plsc.unpack(
                            v, format=plsc.PackFormat.INTERLEAVED)
                        rv[r, pl.ds(half, 32)] = plsc.pack(
                            va * w, vb * w,
                            format=plsc.PackFormat.INTERLEAVED)
                return 0
            lax.fori_loop(0, SUB // 16, body, 0)

        def one_pass(p, _):
            et = core * 2 + p // NCP
            c = p % NCP
            pltpu.sync_copy(z_hbm.at[pl.ds(sub * RPT, RPT)],
                            acc_sh.at[pl.ds(sub * RPT, RPT)])
            plsc.subcore_barrier()

            def chunk(ch, _):
                f0 = et * EP + sub * EPT + ch * S
                pltpu.sync_copy(src_hbm.at[pl.ds(f0, S)], src_v)
                pltpu.sync_copy(w_hbm.at[pl.ds(f0, S)], w_v)
                drow = pl.multiple_of(f0 // SUB, 8)
                pltpu.sync_copy(dst_hbm.at[pl.ds(drow, NSUB)], dst_v)
                boff = core * (N * NCP) + c
                def adj(t, _):
                    v = src_v[pl.ds(t * 16, 16)]
                    src_v[pl.ds(t * 16, 16)] = v * NCP + boff
                    return 0
                lax.fori_loop(0, S // 16, adj, 0)

                for b in range(NBUF):
                    fire_gather(b, b)

                def step(g, _):
                    for b in range(NBUF):
                        s = g * NBUF + b
                        wait_gather(s, b)
                        scale(b, s)
                        fire_scatter(s, b)
                        wait_scatter(s, b)
                        fire_gather(s + NBUF, b)
                    return 0
                lax.fori_loop(0, NSUB // NBUF - 1, step, 0)

                for b in range(NBUF):
                    s = NSUB - NBUF + b
                    wait_gather(s, b)
                    scale(b, s)
                    pltpu.sync_copy(rows[b], acc_sh.at[dst_v.at[s]], add=True)
                return 0

            lax.fori_loop(0, NCHUNK, chunk, 0)
            plsc.subcore_barrier()
            obase = (et * NCP + c) * N_PAD + sub * RPT
            pltpu.sync_copy(acc_sh.at[pl.ds(sub * RPT, RPT)],
                            out_hbm.at[pl.ds(obase, RPT)])
            return 0

        lax.fori_loop(0, 2 * NCP, one_pass, 0)

    return k(xcat, src_all, dst_all, w_all, zrows)


def _dense_stage(aggs, x, wa, wb, wroot, bvec, wout, bout, et_a, et_b):
    BN = 1024
    grid = (N_PAD // BN,)

    def body(aggA, aggB, x_r, wa_r, wb_r, wr_r, b_r, wo_r, bo_r, o_r):
        h = jnp.dot(x_r[...], wr_r[...], preferred_element_type=jnp.float32)
        h = h + b_r[...]
        for c in range(NCP):
            h = h + jnp.dot(aggA[0, c], wa_r[c],
                            preferred_element_type=jnp.float32)
            h = h + jnp.dot(aggB[0, c], wb_r[c],
                            preferred_element_type=jnp.float32)
        h = jnp.maximum(h, 0.0)
        o = jnp.dot(h, wo_r[...], preferred_element_type=jnp.float32)
        o_r[...] = jnp.maximum(o + bo_r[...], 0.0)

    f = pl.pallas_call(
        body,
        grid=grid,
        in_specs=[
            pl.BlockSpec((1, NCP, BN, CW), lambda i, _e=et_a: (_e, 0, i, 0)),
            pl.BlockSpec((1, NCP, BN, CW), lambda i, _e=et_b: (_e, 0, i, 0)),
            pl.BlockSpec((BN, D), lambda i: (i, 0)),
            pl.BlockSpec((NCP, CW, D), lambda i: (0, 0, 0)),
            pl.BlockSpec((NCP, CW, D), lambda i: (0, 0, 0)),
            pl.BlockSpec((D, D), lambda i: (0, 0)),
            pl.BlockSpec((1, D), lambda i: (0, 0)),
            pl.BlockSpec((D, OUT), lambda i: (0, 0)),
            pl.BlockSpec((1, OUT), lambda i: (0, 0)),
        ],
        out_specs=pl.BlockSpec((BN, OUT), lambda i: (i, 0)),
        out_shape=jax.ShapeDtypeStruct((N, OUT), jnp.float32),
    )
    return f(aggs, aggs, x, wa, wb, wroot, bvec, wout, bout)


def kernel(x_general, x_rainfall, ei_gen_to_gen, ei_gen_to_rain,
           ei_rain_to_gen, ei_rain_to_rain, ew_gen_to_gen, ew_gen_to_rain,
           ew_rain_to_gen, ew_rain_to_rain, params):
    f32 = jnp.float32
    bf16 = jnp.bfloat16

    xcat = jnp.concatenate([x_general, x_rainfall], axis=0).astype(
        bf16).reshape(NC * NCP * N, CW)

    pad = EP - E
    srcs, dsts, ws = [], [], []
    for ei, ew in ((ei_gen_to_gen, ew_gen_to_gen),
                   (ei_gen_to_rain, ew_gen_to_rain),
                   (ei_rain_to_gen, ew_rain_to_gen),
                   (ei_rain_to_rain, ew_rain_to_rain)):
        srcs.append(jnp.concatenate([ei[0], jnp.zeros((pad,), jnp.int32)]))
        dsts.append(jnp.concatenate([ei[1], jnp.full((pad,), N, jnp.int32)]))
        ws.append(jnp.concatenate([ew, jnp.zeros((pad,), f32)]))
    src_all = jnp.concatenate(srcs)
    dst_all = jnp.concatenate(dsts).reshape(4 * EP // SUB, SUB)
    w_all = jnp.concatenate(ws)
    zrows = jnp.zeros((N_PAD, CW), bf16)

    aggs = _sc_segment_sums(xcat, src_all, dst_all, w_all, zrows)
    aggs = aggs.reshape(4, NCP, N_PAD, CW)

    lp = params['layers'][-1]

    def rel_w(p):
        return p['W_rel'].T.reshape(NCP, CW, D).astype(bf16)

    outs = []
    for et_a, et_b, pa, pb, x, lin in (
            (0, 2, lp['gg'], lp['rg'], x_general, params['lin_general']),
            (1, 3, lp['gr'], lp['rr'], x_rainfall, params['lin_rainfall'])):
        wroot = (pa['W_root'] + pb['W_root']).T
        bvec = (pa['b_rel'] + pb['b_rel']).reshape(1, D)
        outs.append(_dense_stage(aggs, x, rel_w(pa), rel_w(pb), wroot, bvec,
                                 lin['W'].T, lin['b'].reshape(1, OUT),
                                 et_a, et_b))

    return (outs[0], outs[1])


# speedup vs baseline: 1.0272x; 1.0160x over previous
import functools

import jax
import jax.numpy as jnp
from jax import lax
from jax.experimental import pallas as pl
from jax.experimental.pallas import tpu as pltpu
from jax.experimental.pallas import tpu_sc as plsc

N = 50000
D = 128
E = 500000
OUT = 64

NC = 2
NS = 16
CW = 64
NCP = D // CW

RPT = 3136
N_PAD = RPT * NS
EPT = 32768
EP = EPT * NS
S = 2048
NCHUNK = EPT // S
SUB = 128
NSUB = S // SUB
NBUF = 4


def _sc_segment_sums(xcat, src_all, dst_all, w_all, zrows):
    mesh = plsc.VectorSubcoreMesh(
        core_axis_name="c", subcore_axis_name="s", num_cores=NC, num_subcores=NS
    )

    @functools.partial(
        pl.kernel,
        out_type=jax.ShapeDtypeStruct((4 * NCP * N_PAD, CW), jnp.bfloat16),
        mesh=mesh,
        scratch_types=[
            [pltpu.VMEM((S,), jnp.int32)] * 2,
            [pltpu.VMEM((NSUB, SUB), jnp.int32)] * 2,
            [pltpu.VMEM((S,), jnp.float32)] * 2,
            [pltpu.VMEM((SUB, CW), jnp.bfloat16)] * NBUF,
            pltpu.VMEM_SHARED((N_PAD, CW), jnp.bfloat16),
            [pltpu.SemaphoreType.DMA] * NBUF,
            [pltpu.SemaphoreType.DMA] * NBUF,
            [pltpu.SemaphoreType.DMA] * 6,
        ],
        compiler_params=pltpu.CompilerParams(use_tc_tiling_on_sc=False,
                                            needs_layout_passes=False),
    )
    def k(xcat_hbm, src_hbm, dst_hbm, w_hbm, z_hbm, out_hbm,
          src_vs, dst_vs, w_vs, rows, acc_sh, gsems, ssems, stsems):
        core = lax.axis_index("c")
        sub = lax.axis_index("s")

        def fire_gather(src_v, s, b):
            pltpu.async_copy(
                xcat_hbm.at[src_v.at[pl.ds(pl.multiple_of(s * SUB, SUB), SUB)]],
                rows[b], gsems[b])

        def wait_gather(src_v, s, b):
            pltpu.make_async_copy(
                xcat_hbm.at[src_v.at[pl.ds(pl.multiple_of(s * SUB, SUB), SUB)]],
                rows[b], gsems[b]).wait()

        def fire_scatter(dst_v, s, b):
            pltpu.async_copy(rows[b], acc_sh.at[dst_v.at[s]], ssems[b],
                             add=True)

        def wait_scatter(dst_v, s, b):
            pltpu.make_async_copy(rows[b], acc_sh.at[dst_v.at[s]],
                                  ssems[b]).wait()

        def fire_stage(cs, f0):
            pltpu.async_copy(src_hbm.at[pl.ds(f0, S)], src_vs[cs],
                             stsems[3 * cs])
            pltpu.async_copy(w_hbm.at[pl.ds(f0, S)], w_vs[cs],
                             stsems[3 * cs + 1])
            drow = pl.multiple_of(f0 // SUB, 8)
            pltpu.async_copy(dst_hbm.at[pl.ds(drow, NSUB)], dst_vs[cs],
                             stsems[3 * cs + 2])

        def wait_stage(cs, f0):
            pltpu.make_async_copy(src_hbm.at[pl.ds(f0, S)], src_vs[cs],
                                  stsems[3 * cs]).wait()
            pltpu.make_async_copy(w_hbm.at[pl.ds(f0, S)], w_vs[cs],
                                  stsems[3 * cs + 1]).wait()
            drow = pl.multiple_of(f0 // SUB, 8)
            pltpu.make_async_copy(dst_hbm.at[pl.ds(drow, NSUB)], dst_vs[cs],
                                  stsems[3 * cs + 2]).wait()

        def scale(w_v, b, s):
            rv = rows[b]
            def body(gi, _):
                w0 = pl.multiple_of(s * SUB + gi * 16, 16)
                wv = w_v[pl.ds(w0, 16)]
                for t in range(16):
                    w = wv[t]
                    r = gi * 16 + t
                    for half in (0, 32):
                        v = rv[r, pl.ds(half, 32)]
                        va, vb = plsc.unpack(
                            v, format=plsc.PackFormat.INTERLEAVED)
                        rv[r, pl.ds(half, 32)] = plsc.pack(
                            va * w, vb * w,
                            format=plsc.PackFormat.INTERLEAVED)
                return 0
            lax.fori_loop(0, SUB // 16, body, 0)

        def one_pass(p, _):
            et = core * 2 + p // NCP
            c = p % NCP
            pltpu.sync_copy(z_hbm.at[pl.ds(sub * RPT, RPT)],
                            acc_sh.at[pl.ds(sub * RPT, RPT)])
            plsc.subcore_barrier()

            ebase = et * EP + sub * EPT
            boff = core * (N * NCP) + c

            def ring(cs, ch):
                src_v, dst_v, w_v = src_vs[cs], dst_vs[cs], w_vs[cs]
                wait_stage(cs, ebase + ch * S)
                def adj(t, _):
                    v = src_v[pl.ds(t * 16, 16)]
                    src_v[pl.ds(t * 16, 16)] = v * NCP + boff
                    return 0
                lax.fori_loop(0, S // 16, adj, 0)
                @pl.when(ch + 1 < NCHUNK)
                def _():
                    fire_stage(1 - cs, ebase + (ch + 1) * S)

                for b in range(NBUF):
                    fire_gather(src_v, b, b)

                def step(g, _):
                    for b in range(NBUF):
                        s = g * NBUF + b
                        wait_gather(src_v, s, b)
                        scale(w_v, b, s)
                        fire_scatter(dst_v, s, b)
                        wait_scatter(dst_v, s, b)
                        fire_gather(src_v, s + NBUF, b)
                    return 0
                lax.fori_loop(0, NSUB // NBUF - 1, step, 0)

                for b in range(NBUF):
                    s = NSUB - NBUF + b
                    wait_gather(src_v, s, b)
                    scale(w_v, b, s)
                    pltpu.sync_copy(rows[b], acc_sh.at[dst_v.at[s]], add=True)

            fire_stage(0, ebase)
            def dchunk(g, _):
                ring(0, 2 * g)
                ring(1, 2 * g + 1)
                return 0
            lax.fori_loop(0, NCHUNK // 2, dchunk, 0)
            plsc.subcore_barrier()
            obase = (et * NCP + c) * N_PAD + sub * RPT
            pltpu.sync_copy(acc_sh.at[pl.ds(sub * RPT, RPT)],
                            out_hbm.at[pl.ds(obase, RPT)])
            return 0

        lax.fori_loop(0, 2 * NCP, one_pass, 0)

    return k(xcat, src_all, dst_all, w_all, zrows)


def _dense_stage(aggs, x, wa, wb, wroot, bvec, wout, bout, et_a, et_b):
    BN = 1024
    grid = (N_PAD // BN,)

    def body(aggA, aggB, x_r, wa_r, wb_r, wr_r, b_r, wo_r, bo_r, o_r):
        h = jnp.dot(x_r[...], wr_r[...], preferred_element_type=jnp.float32)
        h = h + b_r[...]
        for c in range(NCP):
            h = h + jnp.dot(aggA[0, c], wa_r[c],
                            preferred_element_type=jnp.float32)
            h = h + jnp.dot(aggB[0, c], wb_r[c],
                            preferred_element_type=jnp.float32)
        h = jnp.maximum(h, 0.0)
        o = jnp.dot(h, wo_r[...], preferred_element_type=jnp.float32)
        o_r[...] = jnp.maximum(o + bo_r[...], 0.0)

    f = pl.pallas_call(
        body,
        grid=grid,
        in_specs=[
            pl.BlockSpec((1, NCP, BN, CW), lambda i, _e=et_a: (_e, 0, i, 0)),
            pl.BlockSpec((1, NCP, BN, CW), lambda i, _e=et_b: (_e, 0, i, 0)),
            pl.BlockSpec((BN, D), lambda i: (i, 0)),
            pl.BlockSpec((NCP, CW, D), lambda i: (0, 0, 0)),
            pl.BlockSpec((NCP, CW, D), lambda i: (0, 0, 0)),
            pl.BlockSpec((D, D), lambda i: (0, 0)),
            pl.BlockSpec((1, D), lambda i: (0, 0)),
            pl.BlockSpec((D, OUT), lambda i: (0, 0)),
            pl.BlockSpec((1, OUT), lambda i: (0, 0)),
        ],
        out_specs=pl.BlockSpec((BN, OUT), lambda i: (i, 0)),
        out_shape=jax.ShapeDtypeStruct((N, OUT), jnp.float32),
    )
    return f(aggs, aggs, x, wa, wb, wroot, bvec, wout, bout)


def kernel(x_general, x_rainfall, ei_gen_to_gen, ei_gen_to_rain,
           ei_rain_to_gen, ei_rain_to_rain, ew_gen_to_gen, ew_gen_to_rain,
           ew_rain_to_gen, ew_rain_to_rain, params):
    f32 = jnp.float32
    bf16 = jnp.bfloat16

    xcat = jnp.concatenate([x_general, x_rainfall], axis=0).astype(
        bf16).reshape(NC * NCP * N, CW)

    pad = EP - E
    srcs, dsts, ws = [], [], []
    for ei, ew in ((ei_gen_to_gen, ew_gen_to_gen),
                   (ei_gen_to_rain, ew_gen_to_rain),
                   (ei_rain_to_gen, ew_rain_to_gen),
                   (ei_rain_to_rain, ew_rain_to_rain)):
        srcs.append(jnp.concatenate([ei[0], jnp.zeros((pad,), jnp.int32)]))
        dsts.append(jnp.concatenate([ei[1], jnp.full((pad,), N, jnp.int32)]))
        ws.append(jnp.concatenate([ew, jnp.zeros((pad,), f32)]))
    src_all = jnp.concatenate(srcs)
    dst_all = jnp.concatenate(dsts).reshape(4 * EP // SUB, SUB)
    w_all = jnp.concatenate(ws)
    zrows = jnp.zeros((N_PAD, CW), bf16)

    aggs = _sc_segment_sums(xcat, src_all, dst_all, w_all, zrows)
    aggs = aggs.reshape(4, NCP, N_PAD, CW)

    lp = params['layers'][-1]

    def rel_w(p):
        return p['W_rel'].T.reshape(NCP, CW, D).astype(bf16)

    outs = []
    for et_a, et_b, pa, pb, x, lin in (
            (0, 2, lp['gg'], lp['rg'], x_general, params['lin_general']),
            (1, 3, lp['gr'], lp['rr'], x_rainfall, params['lin_rainfall'])):
        wroot = (pa['W_root'] + pb['W_root']).T
        bvec = (pa['b_rel'] + pb['b_rel']).reshape(1, D)
        outs.append(_dense_stage(aggs, x, rel_w(pa), rel_w(pb), wroot, bvec,
                                 lin['W'].T, lin['b'].reshape(1, OUT),
                                 et_a, et_b))

    return (outs[0], outs[1])
